# Initial kernel scaffold; baseline (speedup 1.0000x reference)
import jax
import jax.numpy as jnp
from jax.experimental import pallas as pl


def kernel(pcs):
    raise NotImplementedError("write your pallas kernel here")


# speedup vs baseline: 373.8304x; 373.8304x over previous
import functools

import jax
import jax.numpy as jnp
from jax import lax
from jax.experimental import pallas as pl
from jax.experimental.pallas import tpu as pltpu
from jax.experimental.pallas import tpu_sc as plsc

B = 32
N = 16384
S = 64
L = 16
NV = N // L
RADIUS_SQ = 1.0

_mesh = plsc.VectorSubcoreMesh(core_axis_name="c", subcore_axis_name="s")


@functools.partial(
    pl.kernel,
    mesh=_mesh,
    compiler_params=pltpu.CompilerParams(needs_layout_passes=False),
    out_type=jax.ShapeDtypeStruct((B, 8, S), jnp.int32),
    scratch_types=[
        pltpu.VMEM((3, N), jnp.float32),
        pltpu.VMEM((8, S), jnp.int32),
    ],
)
def _octant_query_sc(pcs_hbm, out_hbm, pts, outbuf):
    wid = lax.axis_index("s") * 2 + lax.axis_index("c")

    pltpu.sync_copy(pcs_hbm.at[wid], pts)

    neg1 = jnp.full((L,), -1, jnp.int32)
    for o in range(8):
        for j in range(S // L):
            outbuf[o, pl.ds(j * L, L)] = neg1

    zeros = jnp.zeros((L,), jnp.int32)
    init_counts = (zeros,) * 8
    lane = lax.iota(jnp.int32, L)

    def cond(carry):
        i, counts = carry
        full = (counts[0] >= S)
        for o in range(1, 8):
            full = full & (counts[o] >= S)
        return (i < NV) & jnp.logical_not(jnp.all(full))

    def body(carry):
        i, counts = carry
        off = i * L
        x = pts[0, pl.ds(off, L)]
        y = pts[1, pl.ds(off, L)]
        z = pts[2, pl.ds(off, L)]
        oct_id = ((x > 0).astype(jnp.int32) * 4
                  + (y > 0).astype(jnp.int32) * 2
                  + (z > 0).astype(jnp.int32))
        within = (x * x + y * y + z * z) <= RADIUS_SQ
        idx_vec = lane + off
        new_counts = []
        for o in range(8):
            m = within & (oct_id == o)
            incl = plsc.cumsum(m.astype(jnp.int32))
            slot = counts[o] + incl - 1
            sel = m & (slot < S)
            slot_c = jnp.where(sel, slot, 0)
            plsc.store_scatter(
                outbuf, [jnp.full((L,), o, jnp.int32), slot_c], idx_vec,
                mask=sel)
            pop = plsc.all_reduce_population_count(m)
            new_counts.append(counts[o] + pop)
        return i + 1, tuple(new_counts)

    lax.while_loop(cond, body, (jnp.int32(0), init_counts))

    pltpu.sync_copy(outbuf, out_hbm.at[wid])


def kernel(pcs):
    return _octant_query_sc(pcs)


# speedup vs baseline: 394.1555x; 1.0544x over previous
import functools

import jax
import jax.numpy as jnp
from jax import lax
from jax.experimental import pallas as pl
from jax.experimental.pallas import tpu as pltpu
from jax.experimental.pallas import tpu_sc as plsc

B = 32
N = 16384
S = 64
L = 16
NV = N // L
W = S + L
RADIUS_SQ = 1.0

_mesh = plsc.VectorSubcoreMesh(core_axis_name="c", subcore_axis_name="s")


@functools.partial(
    pl.kernel,
    mesh=_mesh,
    compiler_params=pltpu.CompilerParams(needs_layout_passes=False),
    out_type=jax.ShapeDtypeStruct((B, 8, S), jnp.int32),
    scratch_types=[
        pltpu.VMEM((3, N), jnp.float32),
        pltpu.VMEM((8, W), jnp.int32),
    ],
)
def _octant_query_sc(pcs_hbm, out_hbm, pts, obuf):
    wid = lax.axis_index("s") * 2 + lax.axis_index("c")

    pltpu.sync_copy(pcs_hbm.at[wid], pts)

    neg1 = jnp.full((L,), -1, jnp.int32)
    for o in range(8):
        for j in range(W // L):
            obuf[o, pl.ds(j * L, L)] = neg1

    lane = lax.iota(jnp.int32, L)
    zero = jnp.int32(0)

    def cond(carry):
        i, ptrs = carry
        mn = jnp.minimum(jnp.minimum(jnp.minimum(ptrs[0], ptrs[1]),
                                     jnp.minimum(ptrs[2], ptrs[3])),
                         jnp.minimum(jnp.minimum(ptrs[4], ptrs[5]),
                                     jnp.minimum(ptrs[6], ptrs[7])))
        return (i < NV) & (mn < S)

    def body(carry):
        i, ptrs = carry
        off = i * L
        x = pts[0, pl.ds(off, L)]
        y = pts[1, pl.ds(off, L)]
        z = pts[2, pl.ds(off, L)]
        oct_id = ((x > 0).astype(jnp.int32) * 4
                  + (y > 0).astype(jnp.int32) * 2
                  + (z > 0).astype(jnp.int32))
        within = (x * x + y * y + z * z) <= RADIUS_SQ
        idx_vec = lane + off
        new_ptrs = []
        for o in range(8):
            m = within & (oct_id == o)
            plsc.store_compressed(obuf.at[o, pl.ds(ptrs[o], L)], idx_vec,
                                  mask=m)
            pop = jnp.sum(m.astype(jnp.int32))
            new_ptrs.append(jnp.minimum(ptrs[o] + pop, S))
        return i + 1, tuple(new_ptrs)

    lax.while_loop(cond, body, (zero, (zero,) * 8))

    for o in range(8):
        pltpu.sync_copy(obuf.at[o, pl.ds(0, S)], out_hbm.at[wid, o])


def kernel(pcs):
    return _octant_query_sc(pcs)
